# Initial kernel scaffold; baseline (speedup 1.0000x reference)
import jax
import jax.numpy as jnp
from jax.experimental import pallas as pl


def kernel(x, batch, W_ih, W_hh, b_ih, b_hh):
    raise NotImplementedError("write your pallas kernel here")


# speedup vs baseline: 4.1888x; 4.1888x over previous
import functools

import jax
import jax.numpy as jnp
import numpy as np
from jax import lax
from jax.experimental import pallas as pl
from jax.experimental.pallas import tpu as pltpu
from jax.experimental.pallas import tpu_sc as plsc

N = 50000
D = 256
B = 64
HID = 256
STEPS = 3
CHUNK = 80
NCHUNK = N // CHUNK
NW = 32
ITERS = (NCHUNK + NW - 1) // NW
NK = D // 16
ACC_W = D + 16


def _attn_body(x_hbm, batch_hbm, q_hbm, out_hbm, qv, acc, xbuf, bbuf):
    wid = lax.axis_index("s") * 2 + lax.axis_index("c")

    pltpu.sync_copy(q_hbm, qv)

    zero = jnp.zeros((16,), jnp.float32)

    def zero_body(i, carry):
        for k in range(ACC_W // 16):
            acc[i, pl.ds(k * 16, 16)] = zero
        return carry

    lax.fori_loop(0, B, zero_body, 0)

    def chunk_body(i, carry):
        cid = wid + NW * i

        @pl.when(cid < NCHUNK)
        def _():
            start = cid * CHUNK
            pltpu.sync_copy(x_hbm.at[pl.ds(start, CHUNK)], xbuf)
            pltpu.sync_copy(batch_hbm.at[pl.ds(start, CHUNK)],
                            bbuf.at[pl.ds(0, CHUNK)])

            def row_body(j, rcarry):
                b = bbuf[pl.ds(j, 16)][0]
                xs = [xbuf[j, pl.ds(k * 16, 16)] for k in range(NK)]
                dot = xs[0] * qv[b, pl.ds(0, 16)]
                for k in range(1, NK):
                    dot = dot + xs[k] * qv[b, pl.ds(k * 16, 16)]
                lane = lax.iota(jnp.int32, 16)
                for s in (8, 4, 2, 1):
                    dot = dot + jnp.take_along_axis(
                        dot, lane ^ s, axis=0, mode="promise_in_bounds")
                av = jnp.minimum(jnp.maximum(dot, -20.0), 20.0)
                ev = jnp.exp(av)
                for k in range(NK):
                    plsc.addupdate(acc.at[b, pl.ds(k * 16, 16)], xs[k] * ev)
                plsc.addupdate(acc.at[b, pl.ds(D, 16)], ev)
                return rcarry

            lax.fori_loop(0, CHUNK, row_body, 0)

        return carry

    lax.fori_loop(0, ITERS, chunk_body, 0)

    pltpu.sync_copy(acc, out_hbm.at[wid])


_attn = pl.kernel(
    _attn_body,
    out_type=jax.ShapeDtypeStruct((NW, B, ACC_W), jnp.float32),
    mesh=plsc.VectorSubcoreMesh(core_axis_name="c", subcore_axis_name="s"),
    scratch_types=[
        pltpu.VMEM((B, D), jnp.float32),
        pltpu.VMEM((B, ACC_W), jnp.float32),
        pltpu.VMEM((CHUNK, D), jnp.float32),
        pltpu.VMEM((CHUNK + 16,), jnp.int32),
    ],
)


def _lstm_body(part_ref, qprev_ref, h_ref, c_ref, wih_ref, whh_ref, bias_ref,
               h_out, c_out, qs_out):
    acc = jnp.sum(part_ref[...], axis=0)
    se = acc[:, D:D + 1]
    r = acc[:, :D] / (se + 1e-8)
    qs = jnp.concatenate([qprev_ref[...], r], axis=1)
    gates = lax.dot_general(qs, wih_ref[...], (((1,), (1,)), ((), ())),
                            preferred_element_type=jnp.float32)
    gates = gates + lax.dot_general(h_ref[...], whh_ref[...],
                                    (((1,), (1,)), ((), ())),
                                    preferred_element_type=jnp.float32)
    gates = gates + bias_ref[...]
    i = jax.nn.sigmoid(gates[:, 0:HID])
    f = jax.nn.sigmoid(gates[:, HID:2 * HID])
    g = jnp.tanh(gates[:, 2 * HID:3 * HID])
    o = jax.nn.sigmoid(gates[:, 3 * HID:4 * HID])
    c_new = f * c_ref[...] + i * g
    h_new = o * jnp.tanh(c_new)
    h_out[...] = h_new
    c_out[...] = c_new
    qs_out[...] = qs


_lstm = pl.pallas_call(
    _lstm_body,
    out_shape=(
        jax.ShapeDtypeStruct((B, HID), jnp.float32),
        jax.ShapeDtypeStruct((B, HID), jnp.float32),
        jax.ShapeDtypeStruct((B, 2 * D), jnp.float32),
    ),
)


def kernel(x, batch, W_ih, W_hh, b_ih, b_hh):
    bias = (b_ih + b_hh).reshape(1, 4 * HID)
    part = jnp.zeros((NW, B, ACC_W), jnp.float32)
    h = jnp.zeros((B, HID), jnp.float32)
    c = jnp.zeros((B, HID), jnp.float32)
    qprev = jnp.zeros((B, HID), jnp.float32)
    for _ in range(STEPS):
        h, c, _ = _lstm(part, qprev, h, c, W_ih, W_hh, bias)
        qprev = h
        part = _attn(x, batch, h)
    _, _, q_star = _lstm(part, qprev, h, c, W_ih, W_hh, bias)
    return q_star


# speedup vs baseline: 5.9134x; 1.4117x over previous
import jax
import jax.numpy as jnp
from jax import lax
from jax.experimental import pallas as pl
from jax.experimental.pallas import tpu as pltpu
from jax.experimental.pallas import tpu_sc as plsc

N = 50000
D = 256
B = 64
HID = 256
STEPS = 3
CHUNK = 80
NCHUNK = N // CHUNK
NW = 32
ITERS = (NCHUNK + NW - 1) // NW
NK = D // 16
ACC_W = D + 16


def _reduce_bcast(dot):
    lane = lax.iota(jnp.int32, 16)
    for s in (8, 4, 2, 1):
        dot = dot + jnp.take_along_axis(dot, lane ^ s, axis=0,
                                        mode="promise_in_bounds")
    return dot


def _row_attn(xs, qr):
    ps = []
    for g in range(4):
        p = xs[4 * g] * qr[4 * g]
        for k in range(4 * g + 1, 4 * g + 4):
            p = p + xs[k] * qr[k]
        ps.append(p)
    dot = (ps[0] + ps[1]) + (ps[2] + ps[3])
    av = _reduce_bcast(dot)
    av = jnp.minimum(jnp.maximum(av, -20.0), 20.0)
    return jnp.exp(av)


def _attn_body(x_hbm, batch_hbm, q_hbm, out_hbm, qv, acc, xbuf, bbuf,
               xsem0, xsem1, bsem0, bsem1):
    wid = lax.axis_index("s") * 2 + lax.axis_index("c")
    xsems = (xsem0, xsem1)
    bsems = (bsem0, bsem1)

    pltpu.sync_copy(q_hbm, qv)

    zero = jnp.zeros((16,), jnp.float32)

    def zero_body(i, carry):
        for k in range(ACC_W // 16):
            acc[i, pl.ds(k * 16, 16)] = zero
        return carry

    lax.fori_loop(0, B, zero_body, 0)

    def start_chunk(cid, slot):
        @pl.when(cid < NCHUNK)
        def _():
            st = cid * CHUNK
            pltpu.make_async_copy(x_hbm.at[pl.ds(st, CHUNK)],
                                  xbuf.at[slot], xsems[slot]).start()
            pltpu.make_async_copy(batch_hbm.at[pl.ds(st, CHUNK)],
                                  bbuf.at[slot, pl.ds(0, CHUNK)],
                                  bsems[slot]).start()

    def wait_chunk(cid, slot):
        @pl.when(cid < NCHUNK)
        def _():
            st = cid * CHUNK
            pltpu.make_async_copy(x_hbm.at[pl.ds(st, CHUNK)],
                                  xbuf.at[slot], xsems[slot]).wait()
            pltpu.make_async_copy(batch_hbm.at[pl.ds(st, CHUNK)],
                                  bbuf.at[slot, pl.ds(0, CHUNK)],
                                  bsems[slot]).wait()

    def process(cid, slot):
        @pl.when(cid < NCHUNK)
        def _():
            b0 = bbuf[slot, pl.ds(0, 16)][0]
            b1 = bbuf[slot, pl.ds(CHUNK - 16, 16)][15]

            @pl.when(b0 == b1)
            def _fast():
                qr = [qv[b0, pl.ds(k * 16, 16)] for k in range(NK)]

                def row(j, carry):
                    xs = [xbuf[slot, j, pl.ds(k * 16, 16)]
                          for k in range(NK)]
                    ev = _row_attn(xs, qr)
                    new = tuple(carry[k] + xs[k] * ev for k in range(NK))
                    return new + (carry[NK] + ev,)

                init = (zero,) * (NK + 1)
                res = lax.fori_loop(0, CHUNK, row, init, unroll=2)
                for k in range(NK):
                    plsc.addupdate(acc.at[b0, pl.ds(k * 16, 16)], res[k])
                plsc.addupdate(acc.at[b0, pl.ds(D, 16)], res[NK])

            @pl.when(b0 != b1)
            def _slow():
                def row(j, rcarry):
                    b = bbuf[slot, pl.ds(j, 16)][0]
                    xs = [xbuf[slot, j, pl.ds(k * 16, 16)]
                          for k in range(NK)]
                    qr = [qv[b, pl.ds(k * 16, 16)] for k in range(NK)]
                    ev = _row_attn(xs, qr)
                    for k in range(NK):
                        plsc.addupdate(acc.at[b, pl.ds(k * 16, 16)],
                                       xs[k] * ev)
                    plsc.addupdate(acc.at[b, pl.ds(D, 16)], ev)
                    return rcarry

                lax.fori_loop(0, CHUNK, row, 0)

    start_chunk(wid, 0)

    def pair_body(p, carry):
        cid0 = wid + NW * (2 * p)
        cid1 = wid + NW * (2 * p + 1)
        cid2 = wid + NW * (2 * p + 2)
        wait_chunk(cid0, 0)
        start_chunk(cid1, 1)
        process(cid0, 0)
        wait_chunk(cid1, 1)
        start_chunk(cid2, 0)
        process(cid1, 1)
        return carry

    lax.fori_loop(0, ITERS // 2, pair_body, 0)

    pltpu.sync_copy(acc, out_hbm.at[wid])


_attn = pl.kernel(
    _attn_body,
    out_type=jax.ShapeDtypeStruct((NW, B, ACC_W), jnp.float32),
    mesh=plsc.VectorSubcoreMesh(core_axis_name="c", subcore_axis_name="s"),
    scratch_types=[
        pltpu.VMEM((B, D), jnp.float32),
        pltpu.VMEM((B, ACC_W), jnp.float32),
        pltpu.VMEM((2, CHUNK, D), jnp.float32),
        pltpu.VMEM((2, CHUNK + 16), jnp.int32),
        pltpu.SemaphoreType.DMA,
        pltpu.SemaphoreType.DMA,
        pltpu.SemaphoreType.DMA,
        pltpu.SemaphoreType.DMA,
    ],
)


def _lstm_body(part_ref, qprev_ref, h_ref, c_ref, wih_ref, whh_ref, bias_ref,
               h_out, c_out, qs_out):
    acc = jnp.sum(part_ref[...], axis=0)
    se = acc[:, D:D + 1]
    r = acc[:, :D] / (se + 1e-8)
    qs = jnp.concatenate([qprev_ref[...], r], axis=1)
    gates = lax.dot_general(qs, wih_ref[...], (((1,), (1,)), ((), ())),
                            preferred_element_type=jnp.float32)
    gates = gates + lax.dot_general(h_ref[...], whh_ref[...],
                                    (((1,), (1,)), ((), ())),
                                    preferred_element_type=jnp.float32)
    gates = gates + bias_ref[...]
    i = jax.nn.sigmoid(gates[:, 0:HID])
    f = jax.nn.sigmoid(gates[:, HID:2 * HID])
    g = jnp.tanh(gates[:, 2 * HID:3 * HID])
    o = jax.nn.sigmoid(gates[:, 3 * HID:4 * HID])
    c_new = f * c_ref[...] + i * g
    h_new = o * jnp.tanh(c_new)
    h_out[...] = h_new
    c_out[...] = c_new
    qs_out[...] = qs


_lstm = pl.pallas_call(
    _lstm_body,
    out_shape=(
        jax.ShapeDtypeStruct((B, HID), jnp.float32),
        jax.ShapeDtypeStruct((B, HID), jnp.float32),
        jax.ShapeDtypeStruct((B, 2 * D), jnp.float32),
    ),
)


def kernel(x, batch, W_ih, W_hh, b_ih, b_hh):
    bias = (b_ih + b_hh).reshape(1, 4 * HID)
    part = jnp.zeros((NW, B, ACC_W), jnp.float32)
    h = jnp.zeros((B, HID), jnp.float32)
    c = jnp.zeros((B, HID), jnp.float32)
    qprev = jnp.zeros((B, HID), jnp.float32)
    for _ in range(STEPS):
        h, c, _ = _lstm(part, qprev, h, c, W_ih, W_hh, bias)
        qprev = h
        part = _attn(x, batch, h)
    _, _, q_star = _lstm(part, qprev, h, c, W_ih, W_hh, bias)
    return q_star


# speedup vs baseline: 6.9103x; 1.1686x over previous
import jax
import jax.numpy as jnp
from jax import lax
from jax.experimental import pallas as pl
from jax.experimental.pallas import tpu as pltpu
from jax.experimental.pallas import tpu_sc as plsc

N = 50000
D = 256
B = 64
HID = 256
STEPS = 3
CHUNK = 80
NCHUNK = N // CHUNK
NW = 32
ITERS = (NCHUNK + NW - 1) // NW
NK = D // 16
ACC_W = D + 16


def _reduce_bcast(dot):
    lane = lax.iota(jnp.int32, 16)
    for s in (8, 4, 2, 1):
        dot = dot + jnp.take_along_axis(dot, lane ^ s, axis=0,
                                        mode="promise_in_bounds")
    return dot


def _row_attn(xs, qr):
    ps = []
    for g in range(4):
        p = xs[4 * g] * qr[4 * g]
        for k in range(4 * g + 1, 4 * g + 4):
            p = p + xs[k] * qr[k]
        ps.append(p)
    dot = (ps[0] + ps[1]) + (ps[2] + ps[3])
    av = _reduce_bcast(dot)
    av = jnp.minimum(jnp.maximum(av, -20.0), 20.0)
    return jnp.exp(av)


def _attn_body(x_hbm, batch_hbm, q_hbm, out_hbm, qv, acc, xbuf, bbuf,
               xsem0, xsem1, bsem0, bsem1):
    wid = lax.axis_index("s") * 2 + lax.axis_index("c")
    xsems = (xsem0, xsem1)
    bsems = (bsem0, bsem1)

    pltpu.sync_copy(q_hbm, qv)

    zero = jnp.zeros((16,), jnp.float32)

    def zero_body(i, carry):
        for k in range(ACC_W // 16):
            acc[i, pl.ds(k * 16, 16)] = zero
        return carry

    lax.fori_loop(0, B, zero_body, 0)

    neg = jnp.full((16,), -1, jnp.int32)
    bbuf[0, pl.ds(CHUNK, 16)] = neg
    bbuf[1, pl.ds(CHUNK, 16)] = neg

    def start_chunk(cid, slot):
        @pl.when(cid < NCHUNK)
        def _():
            st = cid * CHUNK
            pltpu.make_async_copy(x_hbm.at[pl.ds(st, CHUNK)],
                                  xbuf.at[slot], xsems[slot]).start()
            pltpu.make_async_copy(batch_hbm.at[pl.ds(st, CHUNK)],
                                  bbuf.at[slot, pl.ds(0, CHUNK)],
                                  bsems[slot]).start()

    def wait_chunk(cid, slot):
        @pl.when(cid < NCHUNK)
        def _():
            st = cid * CHUNK
            pltpu.make_async_copy(x_hbm.at[pl.ds(st, CHUNK)],
                                  xbuf.at[slot], xsems[slot]).wait()
            pltpu.make_async_copy(batch_hbm.at[pl.ds(st, CHUNK)],
                                  bbuf.at[slot, pl.ds(0, CHUNK)],
                                  bsems[slot]).wait()

    def process(cid, slot):
        @pl.when(cid < NCHUNK)
        def _():
            def window_body(w, carry):
                p = 16 * w
                v = bbuf[slot, pl.ds(p, 16)]
                b0 = v[0]
                b15 = v[15]

                @pl.when(b0 == b15)
                def _fast():
                    qr = [qv[b0, pl.ds(k * 16, 16)] for k in range(NK)]

                    def row(jj, rc):
                        j = p + jj
                        xs = [xbuf[slot, j, pl.ds(k * 16, 16)]
                              for k in range(NK)]
                        ev = _row_attn(xs, qr)
                        for k in range(NK):
                            plsc.addupdate(acc.at[b0, pl.ds(k * 16, 16)],
                                           xs[k] * ev)
                        plsc.addupdate(acc.at[b0, pl.ds(D, 16)], ev)
                        return rc

                    lax.fori_loop(0, 16, row, 0, unroll=2)

                @pl.when(b0 != b15)
                def _slow():
                    def row(jj, rc):
                        j = p + jj
                        b = bbuf[slot, pl.ds(j, 16)][0]
                        xs = [xbuf[slot, j, pl.ds(k * 16, 16)]
                              for k in range(NK)]
                        qr = [qv[b, pl.ds(k * 16, 16)] for k in range(NK)]
                        ev = _row_attn(xs, qr)
                        for k in range(NK):
                            plsc.addupdate(acc.at[b, pl.ds(k * 16, 16)],
                                           xs[k] * ev)
                        plsc.addupdate(acc.at[b, pl.ds(D, 16)], ev)
                        return rc

                    lax.fori_loop(0, 16, row, 0)

                return carry

            lax.fori_loop(0, CHUNK // 16, window_body, 0)

    start_chunk(wid, 0)

    def pair_body(p, carry):
        cid0 = wid + NW * (2 * p)
        cid1 = wid + NW * (2 * p + 1)
        cid2 = wid + NW * (2 * p + 2)
        wait_chunk(cid0, 0)
        start_chunk(cid1, 1)
        process(cid0, 0)
        wait_chunk(cid1, 1)
        start_chunk(cid2, 0)
        process(cid1, 1)
        return carry

    lax.fori_loop(0, ITERS // 2, pair_body, 0)

    pltpu.sync_copy(acc, out_hbm.at[wid])


_attn = pl.kernel(
    _attn_body,
    out_type=jax.ShapeDtypeStruct((NW, B, ACC_W), jnp.float32),
    mesh=plsc.VectorSubcoreMesh(core_axis_name="c", subcore_axis_name="s"),
    scratch_types=[
        pltpu.VMEM((B, D), jnp.float32),
        pltpu.VMEM((B, ACC_W), jnp.float32),
        pltpu.VMEM((2, CHUNK, D), jnp.float32),
        pltpu.VMEM((2, CHUNK + 16), jnp.int32),
        pltpu.SemaphoreType.DMA,
        pltpu.SemaphoreType.DMA,
        pltpu.SemaphoreType.DMA,
        pltpu.SemaphoreType.DMA,
    ],
)


def _lstm_body(part_ref, qprev_ref, h_ref, c_ref, wih_ref, whh_ref, bias_ref,
               h_out, c_out, qs_out):
    acc = jnp.sum(part_ref[...], axis=0)
    se = acc[:, D:D + 1]
    r = acc[:, :D] / (se + 1e-8)
    qs = jnp.concatenate([qprev_ref[...], r], axis=1)
    gates = lax.dot_general(qs, wih_ref[...], (((1,), (1,)), ((), ())),
                            preferred_element_type=jnp.float32)
    gates = gates + lax.dot_general(h_ref[...], whh_ref[...],
                                    (((1,), (1,)), ((), ())),
                                    preferred_element_type=jnp.float32)
    gates = gates + bias_ref[...]
    i = jax.nn.sigmoid(gates[:, 0:HID])
    f = jax.nn.sigmoid(gates[:, HID:2 * HID])
    g = jnp.tanh(gates[:, 2 * HID:3 * HID])
    o = jax.nn.sigmoid(gates[:, 3 * HID:4 * HID])
    c_new = f * c_ref[...] + i * g
    h_new = o * jnp.tanh(c_new)
    h_out[...] = h_new
    c_out[...] = c_new
    qs_out[...] = qs


_lstm = pl.pallas_call(
    _lstm_body,
    out_shape=(
        jax.ShapeDtypeStruct((B, HID), jnp.float32),
        jax.ShapeDtypeStruct((B, HID), jnp.float32),
        jax.ShapeDtypeStruct((B, 2 * D), jnp.float32),
    ),
)


def kernel(x, batch, W_ih, W_hh, b_ih, b_hh):
    bias = (b_ih + b_hh).reshape(1, 4 * HID)
    part = jnp.zeros((NW, B, ACC_W), jnp.float32)
    h = jnp.zeros((B, HID), jnp.float32)
    c = jnp.zeros((B, HID), jnp.float32)
    qprev = jnp.zeros((B, HID), jnp.float32)
    for _ in range(STEPS):
        h, c, _ = _lstm(part, qprev, h, c, W_ih, W_hh, bias)
        qprev = h
        part = _attn(x, batch, h)
    _, _, q_star = _lstm(part, qprev, h, c, W_ih, W_hh, bias)
    return q_star
